# speedup vs baseline: 3.9387x; 3.9387x over previous
import functools

import jax
import jax.numpy as jnp
from jax.experimental import pallas as pl


def _moe_dense_body(dff, x_ref, w1_ref, w2_ref, c_ref, o_ref):
    e = pl.program_id(1)
    x = x_ref[...]
    h = jax.lax.dot_general(
        x, w1_ref[e],
        (((1,), (1,)), ((), ())),
        preferred_element_type=jnp.float32,
    )
    gate = h[:, :dff]
    up = h[:, dff:]
    act = (gate * jax.nn.sigmoid(gate) * up).astype(jnp.bfloat16)
    y = jax.lax.dot_general(
        act, w2_ref[e],
        (((1,), (1,)), ((), ())),
        preferred_element_type=jnp.float32,
    )
    c_all = c_ref[...]
    lane = jax.lax.broadcasted_iota(jnp.int32, c_all.shape, 1)
    c = jnp.sum(jnp.where(lane == e, c_all, 0.0), axis=1, keepdims=True)
    contrib = y * c

    @pl.when(e == 0)
    def _init():
        o_ref[...] = contrib

    @pl.when(e > 0)
    def _acc():
        o_ref[...] += contrib


def kernel(hidden_states, w1, w2, topk_weights, topk_ids):
    num_tokens, d = hidden_states.shape
    num_experts = w1.shape[0]
    dff = w2.shape[2]

    ids = topk_ids.astype(jnp.int32)
    coeff = jnp.zeros((num_tokens, num_experts), jnp.float32)
    coeff = coeff.at[jnp.arange(num_tokens)[:, None], ids].add(topk_weights)

    xb = hidden_states.astype(jnp.bfloat16)
    w1b = w1.astype(jnp.bfloat16)
    w2b = w2.astype(jnp.bfloat16)

    bt = 256
    nt = num_tokens // bt
    grid = (nt, num_experts)

    out = pl.pallas_call(
        functools.partial(_moe_dense_body, dff),
        grid=grid,
        in_specs=[
            pl.BlockSpec((bt, d), lambda t, e: (t, 0)),
            pl.BlockSpec((num_experts, 2 * dff, d), lambda t, e: (0, 0, 0)),
            pl.BlockSpec((num_experts, d, dff), lambda t, e: (0, 0, 0)),
            pl.BlockSpec((bt, num_experts), lambda t, e: (t, 0)),
        ],
        out_specs=pl.BlockSpec((bt, d), lambda t, e: (t, 0)),
        out_shape=jax.ShapeDtypeStruct((num_tokens, d), jnp.float32),
    )(xb, w1b, w2b, coeff)
    return out


# speedup vs baseline: 5.5250x; 1.4028x over previous
import functools

import jax
import jax.numpy as jnp
from jax.experimental import pallas as pl


def _moe_dense_body(dff, x_ref, w1_ref, w2_ref, ids_ref, tw_ref, o_ref):
    e = pl.program_id(1)
    x = x_ref[...]
    h = jax.lax.dot_general(
        x, w1_ref[e],
        (((1,), (1,)), ((), ())),
        preferred_element_type=jnp.float32,
    )
    gate = h[:, :dff]
    up = h[:, dff:]
    act = (gate * jax.nn.sigmoid(gate) * up).astype(jnp.bfloat16)
    y = jax.lax.dot_general(
        act, w2_ref[e],
        (((1,), (1,)), ((), ())),
        preferred_element_type=jnp.float32,
    )
    ids = ids_ref[...]
    tw = tw_ref[...]
    c = jnp.sum(jnp.where(ids == e, tw, 0.0), axis=1, keepdims=True)
    contrib = y * c

    @pl.when(e == 0)
    def _init():
        o_ref[...] = contrib

    @pl.when(e > 0)
    def _acc():
        o_ref[...] += contrib


def kernel(hidden_states, w1, w2, topk_weights, topk_ids):
    num_tokens, d = hidden_states.shape
    num_experts = w1.shape[0]
    dff = w2.shape[2]
    k = topk_ids.shape[1]

    ids = topk_ids.astype(jnp.int32)
    xb = hidden_states.astype(jnp.bfloat16)
    w1b = w1.astype(jnp.bfloat16)
    w2b = w2.astype(jnp.bfloat16)

    bt = 256
    nt = num_tokens // bt
    grid = (nt, num_experts)

    out = pl.pallas_call(
        functools.partial(_moe_dense_body, dff),
        grid=grid,
        in_specs=[
            pl.BlockSpec((bt, d), lambda t, e: (t, 0)),
            pl.BlockSpec((num_experts, 2 * dff, d), lambda t, e: (0, 0, 0)),
            pl.BlockSpec((num_experts, d, dff), lambda t, e: (0, 0, 0)),
            pl.BlockSpec((bt, k), lambda t, e: (t, 0)),
            pl.BlockSpec((bt, k), lambda t, e: (t, 0)),
        ],
        out_specs=pl.BlockSpec((bt, d), lambda t, e: (t, 0)),
        out_shape=jax.ShapeDtypeStruct((num_tokens, d), jnp.float32),
    )(xb, w1b, w2b, ids, topk_weights)
    return out
